# baseline (speedup 1.0000x reference)
import jax
import jax.numpy as jnp
from jax import lax
from jax.experimental import pallas as pl
from jax.experimental.pallas import tpu as pltpu
from jax.experimental.pallas import tpu_sc as plsc

N = 10000
E = 320000
D = 128
H = 128
O = 128
ED = 16
B = 64

NC = 2
NS = 16
NW = NC * NS
L = 16
K8 = H // L

EPW = E // NW
C = 40
NCHUNK = EPW // C
ZR = 40


def _sc_edge_body(xl_hbm, xr_hbm, e_hbm, ei_hbm, att_hbm,
                  acc_hbm, den_hbm,
                  acc_sp, xl_v, xr_v, en_v, nsc_v, idx_v, dsc_v, att_v,
                  hs_v, den_v, sem_g, sem_i, sem_d, sem_s):
    cc = lax.axis_index("c")
    ss = lax.axis_index("s")
    wid = ss * NC + cc

    def _zrow(r, _):
        for k in range(K8):
            nsc_v[0][r, pl.ds(k * L, L)] = jnp.zeros((L,), jnp.float32)
        return _
    lax.fori_loop(0, ZR, _zrow, None)
    nblk = jnp.where(ss == NS - 1, 10, 16)
    row0 = ss * 640

    def _zblk(j, _):
        r0 = pl.multiple_of(row0 + j * ZR, ZR)
        pltpu.sync_copy(nsc_v[0], acc_sp.at[pl.ds(r0, ZR)])
        return _
    lax.fori_loop(0, nblk, _zblk, None)

    def _zden(r, _):
        den_v[pl.ds(r * L, L)] = jnp.zeros((L,), jnp.float32)
        return _
    lax.fori_loop(0, N // L, _zden, None)
    pltpu.sync_copy(att_hbm, att_v)
    plsc.subcore_barrier()

    att_regs = [att_v[pl.ds(k * L, L)] for k in range(K8)]
    lanes = lax.broadcasted_iota(jnp.int32, (L,), 0)
    lane0 = lanes == 0

    def _hsum(v):
        for sh in (8, 4, 2, 1):
            hs_v[...] = v
            v = v + plsc.load_gather(hs_v, [lanes ^ sh])
        return v

    def _bf2(ref, i, k):
        x = i * (H // 2) + k * L
        w = ref[jnp.right_shift(x, 7), pl.ds(jnp.bitwise_and(x, 127), L)]
        ev = plsc.bitcast(jnp.left_shift(w, 16), jnp.float32)
        od = plsc.bitcast(jnp.bitwise_and(w, jnp.int32(-65536)), jnp.float32)
        return ev, od

    def _g_issue(b, ch):
        pltpu.async_copy(xl_hbm.at[idx_v[b].at[0]], xl_v[b], sem_g[b])
        pltpu.async_copy(xr_hbm.at[idx_v[b].at[1]], xr_v[b], sem_g[b])
        pltpu.async_copy(e_hbm.at[wid, ch], en_v[b], sem_g[b])

    def _compute(b):
        def _edge(i, _):
            xl_regs = [xl_v[b][i, pl.ds(k * L, L)] for k in range(K8)]
            acc = None
            for k in range(K8 // 2):
                e_e, e_o = _bf2(en_v[b], i, k)
                for h, eh in ((0, e_e), (1, e_o)):
                    z = (xl_regs[2 * k + h]
                         + xr_v[b][i, pl.ds((2 * k + h) * L, L)] + eh)
                    lm = jnp.maximum(z, jnp.float32(0.2) * z)
                    t = lm * att_regs[2 * k + h]
                    acc = t if acc is None else acc + t
            ex = jnp.exp(_hsum(acc))
            for k in range(K8):
                nsc_v[b][i, pl.ds(k * L, L)] = xl_regs[k] * ex
            dsp = plsc.load_gather(idx_v[b], [jnp.full((L,), 1, jnp.int32),
                                              lax.broadcast(i, (L,))])
            plsc.addupdate_scatter(den_v, [dsp], ex, mask=lane0)
            return _
        lax.fori_loop(0, C, _edge, None, unroll=4)

    pltpu.sync_copy(ei_hbm.at[wid, 0], idx_v[0])
    _g_issue(0, 0)
    pltpu.async_copy(ei_hbm.at[wid, 1], idx_v[1], sem_i[1])
    pltpu.async_copy(ei_hbm.at[wid, 0, 1], dsc_v[0], sem_d[0])

    def _iter(it, _):
        for b in (0, 1):
            ch = it * 2 + b
            nb = 1 - b
            pltpu.make_async_copy(xl_hbm.at[pl.ds(0, C)], xl_v[b],
                                  sem_g[b]).wait()
            pltpu.make_async_copy(xr_hbm.at[pl.ds(0, C)], xr_v[b],
                                  sem_g[b]).wait()
            pltpu.make_async_copy(e_hbm.at[0, 0], en_v[b], sem_g[b]).wait()

            @pl.when(ch + 1 < NCHUNK)
            def _():
                pltpu.make_async_copy(ei_hbm.at[0, 0], idx_v[nb],
                                      sem_i[nb]).wait()

                @pl.when(ch >= 1)
                def _():
                    pltpu.make_async_copy(xl_hbm.at[pl.ds(0, C)], nsc_v[nb],
                                          sem_s[nb]).wait()

                _g_issue(nb, ch + 1)
                pltpu.async_copy(ei_hbm.at[wid, ch + 1, 1], dsc_v[nb],
                                 sem_d[nb])

            _compute(b)

            @pl.when(ch + 2 < NCHUNK)
            def _():
                pltpu.async_copy(ei_hbm.at[wid, ch + 2], idx_v[b], sem_i[b])

            pltpu.make_async_copy(ei_hbm.at[0, 0, 1], dsc_v[b],
                                  sem_d[b]).wait()
            pltpu.async_copy(nsc_v[b], acc_sp.at[dsc_v[b]], sem_s[b],
                             add=True)
        return _
    lax.fori_loop(0, NCHUNK // 2, _iter, None)

    pltpu.make_async_copy(xl_hbm.at[pl.ds(0, C)], nsc_v[0], sem_s[0]).wait()
    pltpu.make_async_copy(xl_hbm.at[pl.ds(0, C)], nsc_v[1], sem_s[1]).wait()

    plsc.subcore_barrier()

    def _xblk(j, _):
        r0 = pl.multiple_of(row0 + j * ZR, ZR)
        pltpu.sync_copy(acc_sp.at[pl.ds(r0, ZR)], acc_hbm.at[cc, pl.ds(r0, ZR)])
        return _
    lax.fori_loop(0, nblk, _xblk, None)
    pltpu.sync_copy(den_v, den_hbm.at[wid])


def _pack_i32(a):
    return lax.bitcast_convert_type(a.reshape(a.shape[0], H // 2, 2),
                                    jnp.int32)


def _pack_i32_2n(a):
    return _pack_i32(a).reshape(a.shape[0] // 2, H)


def _sc_edge_layer(xl, xr, e_emb, ei, att):
    e3 = e_emb.reshape(NW, NCHUNK, C // 2, H)
    f = pl.kernel(
        _sc_edge_body,
        out_type=[jax.ShapeDtypeStruct((NC, N, H), jnp.float32),
                  jax.ShapeDtypeStruct((NW, N), jnp.float32)],
        mesh=plsc.VectorSubcoreMesh(core_axis_name="c", subcore_axis_name="s"),
        compiler_params=pltpu.CompilerParams(needs_layout_passes=False),
        scratch_types=[
            pltpu.VMEM_SHARED((N, H), jnp.float32),
            [pltpu.VMEM((C, H), jnp.float32) for _ in range(2)],
            [pltpu.VMEM((C, H), jnp.float32) for _ in range(2)],
            [pltpu.VMEM((C // 2, H), jnp.int32) for _ in range(2)],
            [pltpu.VMEM((C, H), jnp.float32) for _ in range(2)],
            [pltpu.VMEM((3, C), jnp.int32) for _ in range(2)],
            [pltpu.VMEM((C,), jnp.int32) for _ in range(2)],
            pltpu.VMEM((H,), jnp.float32),
            pltpu.VMEM((L,), jnp.float32),
            pltpu.VMEM((N,), jnp.float32),
            [pltpu.SemaphoreType.DMA for _ in range(2)],
            [pltpu.SemaphoreType.DMA for _ in range(2)],
            [pltpu.SemaphoreType.DMA for _ in range(2)],
            [pltpu.SemaphoreType.DMA for _ in range(2)],
        ],
    )
    return f(xl, xr, e3, ei, att)


def _mm2_body(x_ref, w_ref, o1_ref, o2_ref):
    r = jnp.dot(x_ref[...], w_ref[...], preferred_element_type=jnp.float32)
    o1_ref[...] = r[:, :H].astype(o1_ref.dtype)
    o2_ref[...] = r[:, H:].astype(o2_ref.dtype)


def _mm2(x, wcat, bm, dt1, dt2):
    m = x.shape[0]
    k = x.shape[1]
    return pl.pallas_call(
        _mm2_body,
        grid=(m // bm,),
        in_specs=[pl.BlockSpec((bm, k), lambda i: (i, 0)),
                  pl.BlockSpec((k, 2 * H), lambda i: (0, 0))],
        out_specs=[pl.BlockSpec((bm, H), lambda i: (i, 0)),
                   pl.BlockSpec((bm, H), lambda i: (i, 0))],
        out_shape=[jax.ShapeDtypeStruct((m, H), dt1),
                   jax.ShapeDtypeStruct((m, H), dt2)],
    )(x, wcat)


def _combine(acc_ref, den_ref, b_ref):
    a = acc_ref[0] + acc_ref[1]
    den = jnp.sum(den_ref[0], axis=0) + jnp.float32(1e-16)
    return jnp.maximum(a / den[:, None] + b_ref[...], jnp.float32(0.0))


def _mid_body(acc_ref, den_ref, b_ref, w_ref, o1_ref, o2_ref):
    h = _combine(acc_ref, den_ref, b_ref)
    r = jnp.dot(h, w_ref[...], preferred_element_type=jnp.float32)
    o1_ref[...] = r[:, :H].astype(o1_ref.dtype)
    o2_ref[...] = r[:, H:].astype(o2_ref.dtype)


def _mid_layer(acc, den, bias, wcat, bm=2000):
    return pl.pallas_call(
        _mid_body,
        grid=(N // bm,),
        in_specs=[pl.BlockSpec((NC, bm, H), lambda i: (0, i, 0)),
                  pl.BlockSpec((1, NW, bm), lambda i: (i, 0, 0)),
                  pl.BlockSpec((1, H), lambda i: (0, 0)),
                  pl.BlockSpec((H, 2 * H), lambda i: (0, 0))],
        out_specs=[pl.BlockSpec((bm, H), lambda i: (i, 0)),
                   pl.BlockSpec((bm, H), lambda i: (i, 0))],
        out_shape=[jax.ShapeDtypeStruct((N, H), jnp.float32),
                   jax.ShapeDtypeStruct((N, H), jnp.float32)],
    )(acc, den, bias, wcat)


def _final_body(acc_ref, den_ref, b_ref, batch_ref, wfc_ref, bfc_ref, o_ref,
                s_ref, c_ref):
    i = pl.program_id(0)

    @pl.when(i == 0)
    def _():
        s_ref[...] = jnp.zeros_like(s_ref)
        c_ref[...] = jnp.zeros_like(c_ref)

    h = _combine(acc_ref, den_ref, b_ref)
    bids = batch_ref[0, 0, :]
    bm = h.shape[0]
    onehot = (bids[None, :] == lax.broadcasted_iota(jnp.int32, (B, bm), 0)
              ).astype(jnp.float32)
    s_ref[...] += jnp.dot(onehot, h, preferred_element_type=jnp.float32)
    c_ref[...] += jnp.sum(onehot, axis=1)[:, None]

    @pl.when(i == pl.num_programs(0) - 1)
    def _():
        mean = s_ref[...] / jnp.maximum(c_ref[...], jnp.float32(1.0))
        o_ref[...] = (jnp.dot(mean, wfc_ref[...],
                              preferred_element_type=jnp.float32)
                      + bfc_ref[...])


def _final_layer(acc, den, bias, batch3, wfc, bfc, bm=1000):
    return pl.pallas_call(
        _final_body,
        grid=(N // bm,),
        in_specs=[pl.BlockSpec((NC, bm, H), lambda i: (0, i, 0)),
                  pl.BlockSpec((1, NW, bm), lambda i: (i, 0, 0)),
                  pl.BlockSpec((1, H), lambda i: (0, 0)),
                  pl.BlockSpec((1, 1, bm), lambda i: (i, 0, 0)),
                  pl.BlockSpec((H, O), lambda i: (0, 0)),
                  pl.BlockSpec((1, O), lambda i: (0, 0))],
        out_specs=pl.BlockSpec((B, O), lambda i: (0, 0)),
        out_shape=jax.ShapeDtypeStruct((B, O), jnp.float32),
        scratch_shapes=[pltpu.VMEM((B, H), jnp.float32),
                        pltpu.VMEM((B, 1), jnp.float32)],
    )(acc, den, bias, batch3, wfc, bfc)


def kernel(x, edge_index, edge_attr, batch, Wl1, Wr1, We1, att1, b1,
           Wl2, Wr2, We2, att2, b2, Wfc, bfc):
    src3 = edge_index[0].reshape(NW, NCHUNK, C)
    dst3 = edge_index[1].reshape(NW, NCHUNK, C)
    ei = jnp.stack([src3, dst3, jnp.right_shift(dst3, 1)], axis=2)
    batch3 = batch.reshape(N // 1000, 1, 1000)

    blk = jnp.arange(0, H, 32)[:, None]
    evod = jnp.concatenate([jnp.arange(0, 32, 2), jnp.arange(1, 32, 2)])
    P = (blk + evod[None, :]).reshape(H)

    bf = jnp.bfloat16
    xl1, xr1 = _mm2(x, jnp.concatenate([Wl1[:, P], Wr1[:, P]], axis=1), 2000,
                    jnp.float32, jnp.float32)
    e1, e2 = _mm2(edge_attr, jnp.concatenate([We1, We2], axis=1), 2000,
                  bf, bf)

    acc1, den1 = _sc_edge_layer(xl1, xr1, _pack_i32(e1), ei, att1[P])
    den1t = den1.reshape(NW, N // 2000, 2000).transpose(1, 0, 2)
    xl2, xr2 = _mid_layer(acc1, den1t, b1[P].reshape(1, H),
                          jnp.concatenate([Wl2[P][:, P], Wr2[P][:, P]],
                                          axis=1))
    acc2, den2 = _sc_edge_layer(xl2, xr2, _pack_i32(e2), ei, att2[P])
    den2t = den2.reshape(NW, N // 1000, 1000).transpose(1, 0, 2)
    return _final_layer(acc2, den2t, b2[P].reshape(1, H), batch3, Wfc[P],
                        bfc.reshape(1, O))


# speedup vs baseline: 2.0146x; 2.0146x over previous
import jax
import jax.numpy as jnp
from jax import lax
from jax.experimental import pallas as pl
from jax.experimental.pallas import tpu as pltpu
from jax.experimental.pallas import tpu_sc as plsc

N = 10000
E = 320000
D = 128
H = 128
O = 128
ED = 16
B = 64

NC = 2
NS = 16
NW = NC * NS
L = 16
K8 = H // L

EPW = E // NW
C = 40
NCHUNK = EPW // C
ZR = 40


def _sc_edge_body(xl_hbm, xr_hbm, e_hbm, ei_hbm, att_hbm,
                  acc_hbm, den_hbm,
                  acc_sp, xl_v, xr_v, en_v, idx_v, dsc_v, att_v,
                  hs_v, den_v, sem_g, sem_i, sem_d, sem_s):
    cc = lax.axis_index("c")
    ss = lax.axis_index("s")
    wid = ss * NC + cc

    def _zrow(r, _):
        for k in range(K8):
            en_v[0][r, pl.ds(k * L, L)] = jnp.zeros((L,), jnp.float32)
        return _
    lax.fori_loop(0, ZR, _zrow, None)
    nblk = jnp.where(ss == NS - 1, 10, 16)
    row0 = ss * 640

    def _zblk(j, _):
        r0 = pl.multiple_of(row0 + j * ZR, ZR)
        pltpu.sync_copy(en_v[0], acc_sp.at[pl.ds(r0, ZR)])
        return _
    lax.fori_loop(0, nblk, _zblk, None)

    def _zden(r, _):
        den_v[pl.ds(r * L, L)] = jnp.zeros((L,), jnp.float32)
        return _
    lax.fori_loop(0, N // L, _zden, None)
    pltpu.sync_copy(att_hbm, att_v)
    plsc.subcore_barrier()

    att_regs = [att_v[pl.ds(k * L, L)] for k in range(K8)]
    lanes = lax.broadcasted_iota(jnp.int32, (L,), 0)
    lane0 = lanes == 0

    def _hsum(v):
        for sh in (8, 4, 2, 1):
            hs_v[...] = v
            v = v + plsc.load_gather(hs_v, [lanes ^ sh])
        return v

    def _g_issue(b, ch):
        pltpu.async_copy(xl_hbm.at[idx_v[b].at[0]], xl_v[b], sem_g[b])
        pltpu.async_copy(xr_hbm.at[idx_v[b].at[1]], xr_v[b], sem_g[b])
        pltpu.async_copy(e_hbm.at[wid, ch], en_v[b], sem_g[b])

    def _compute(b):
        def _edge(i, _):
            xl_regs = [xl_v[b][i, pl.ds(k * L, L)] for k in range(K8)]
            acc = None
            for k in range(K8):
                z = (xl_regs[k] + xr_v[b][i, pl.ds(k * L, L)]
                     + en_v[b][i, pl.ds(k * L, L)])
                lm = jnp.maximum(z, jnp.float32(0.2) * z)
                t = lm * att_regs[k]
                acc = t if acc is None else acc + t
            ex = jnp.exp(_hsum(acc))
            for k in range(K8):
                en_v[b][i, pl.ds(k * L, L)] = xl_regs[k] * ex
            dsp = plsc.load_gather(idx_v[b], [jnp.full((L,), 1, jnp.int32),
                                              lax.broadcast(i, (L,))])
            plsc.addupdate_scatter(den_v, [dsp], ex, mask=lane0)
            return _
        lax.fori_loop(0, C, _edge, None, unroll=4)

    pltpu.sync_copy(ei_hbm.at[wid, 0], idx_v[0])
    _g_issue(0, 0)
    pltpu.async_copy(ei_hbm.at[wid, 1], idx_v[1], sem_i[1])
    pltpu.async_copy(ei_hbm.at[wid, 0, 1], dsc_v[0], sem_d[0])

    def _iter(it, _):
        for b in (0, 1):
            ch = it * 2 + b
            nb = 1 - b
            pltpu.make_async_copy(e_hbm.at[0, 0], xl_v[b], sem_g[b]).wait()
            pltpu.make_async_copy(e_hbm.at[0, 0], xr_v[b], sem_g[b]).wait()
            pltpu.make_async_copy(e_hbm.at[0, 0], en_v[b], sem_g[b]).wait()

            @pl.when(ch + 1 < NCHUNK)
            def _():
                pltpu.make_async_copy(ei_hbm.at[0, 0], idx_v[nb],
                                      sem_i[nb]).wait()

                @pl.when(ch >= 1)
                def _():
                    pltpu.make_async_copy(e_hbm.at[0, 0], en_v[nb],
                                          sem_s[nb]).wait()

                _g_issue(nb, ch + 1)
                pltpu.async_copy(ei_hbm.at[wid, ch + 1, 1], dsc_v[nb],
                                 sem_d[nb])

            _compute(b)

            @pl.when(ch + 2 < NCHUNK)
            def _():
                pltpu.async_copy(ei_hbm.at[wid, ch + 2], idx_v[b], sem_i[b])

            pltpu.make_async_copy(ei_hbm.at[0, 0, 1], dsc_v[b],
                                  sem_d[b]).wait()
            pltpu.async_copy(en_v[b], acc_sp.at[dsc_v[b]], sem_s[b],
                             add=True)
        return _
    lax.fori_loop(0, NCHUNK // 2, _iter, None)

    pltpu.make_async_copy(e_hbm.at[0, 0], en_v[0], sem_s[0]).wait()
    pltpu.make_async_copy(e_hbm.at[0, 0], en_v[1], sem_s[1]).wait()

    plsc.subcore_barrier()

    def _xblk(j, _):
        r0 = pl.multiple_of(row0 + j * ZR, ZR)
        pltpu.sync_copy(acc_sp.at[pl.ds(r0, ZR)], acc_hbm.at[cc, pl.ds(r0, ZR)])
        return _
    lax.fori_loop(0, nblk, _xblk, None)
    pltpu.sync_copy(den_v, den_hbm.at[wid])


def _sc_edge_layer(xl, xr, e_emb, ei, att):
    e3 = e_emb.reshape(NW, NCHUNK, C, H)
    f = pl.kernel(
        _sc_edge_body,
        out_type=[jax.ShapeDtypeStruct((NC, N, H), jnp.float32),
                  jax.ShapeDtypeStruct((NW, N), jnp.float32)],
        mesh=plsc.VectorSubcoreMesh(core_axis_name="c", subcore_axis_name="s"),
        compiler_params=pltpu.CompilerParams(needs_layout_passes=False),
        scratch_types=[
            pltpu.VMEM_SHARED((N, H), jnp.float32),
            [pltpu.VMEM((C, H), jnp.float32) for _ in range(2)],
            [pltpu.VMEM((C, H), jnp.float32) for _ in range(2)],
            [pltpu.VMEM((C, H), jnp.float32) for _ in range(2)],
            [pltpu.VMEM((2, C), jnp.int32) for _ in range(2)],
            [pltpu.VMEM((C,), jnp.int32) for _ in range(2)],
            pltpu.VMEM((H,), jnp.float32),
            pltpu.VMEM((L,), jnp.float32),
            pltpu.VMEM((N,), jnp.float32),
            [pltpu.SemaphoreType.DMA for _ in range(2)],
            [pltpu.SemaphoreType.DMA for _ in range(2)],
            [pltpu.SemaphoreType.DMA for _ in range(2)],
            [pltpu.SemaphoreType.DMA for _ in range(2)],
        ],
    )
    return f(xl, xr, e3, ei, att)


def _mm2_body(x_ref, w_ref, o1_ref, o2_ref):
    r = jnp.dot(x_ref[...], w_ref[...], preferred_element_type=jnp.float32)
    o1_ref[...] = r[:, :H]
    o2_ref[...] = r[:, H:]


def _mm2(x, wcat, bm):
    m = x.shape[0]
    k = x.shape[1]
    return pl.pallas_call(
        _mm2_body,
        grid=(m // bm,),
        in_specs=[pl.BlockSpec((bm, k), lambda i: (i, 0)),
                  pl.BlockSpec((k, 2 * H), lambda i: (0, 0))],
        out_specs=[pl.BlockSpec((bm, H), lambda i: (i, 0)),
                   pl.BlockSpec((bm, H), lambda i: (i, 0))],
        out_shape=[jax.ShapeDtypeStruct((m, H), jnp.float32),
                   jax.ShapeDtypeStruct((m, H), jnp.float32)],
    )(x, wcat)


def _combine(acc_ref, den_ref, b_ref):
    a = acc_ref[0] + acc_ref[1]
    den = jnp.sum(den_ref[0], axis=0) + jnp.float32(1e-16)
    return jnp.maximum(a / den[:, None] + b_ref[...], jnp.float32(0.0))


def _mid_body(acc_ref, den_ref, b_ref, w_ref, o1_ref, o2_ref):
    h = _combine(acc_ref, den_ref, b_ref)
    r = jnp.dot(h, w_ref[...], preferred_element_type=jnp.float32)
    o1_ref[...] = r[:, :H]
    o2_ref[...] = r[:, H:]


def _mid_layer(acc, den, bias, wcat, bm=1000):
    return pl.pallas_call(
        _mid_body,
        grid=(N // bm,),
        in_specs=[pl.BlockSpec((NC, bm, H), lambda i: (0, i, 0)),
                  pl.BlockSpec((1, NW, bm), lambda i: (i, 0, 0)),
                  pl.BlockSpec((1, H), lambda i: (0, 0)),
                  pl.BlockSpec((H, 2 * H), lambda i: (0, 0))],
        out_specs=[pl.BlockSpec((bm, H), lambda i: (i, 0)),
                   pl.BlockSpec((bm, H), lambda i: (i, 0))],
        out_shape=[jax.ShapeDtypeStruct((N, H), jnp.float32),
                   jax.ShapeDtypeStruct((N, H), jnp.float32)],
    )(acc, den, bias, wcat)


def _final_body(acc_ref, den_ref, b_ref, batch_ref, wfc_ref, bfc_ref, o_ref,
                s_ref, c_ref):
    i = pl.program_id(0)

    @pl.when(i == 0)
    def _():
        s_ref[...] = jnp.zeros_like(s_ref)
        c_ref[...] = jnp.zeros_like(c_ref)

    h = _combine(acc_ref, den_ref, b_ref)
    bids = batch_ref[0, 0, :]
    bm = h.shape[0]
    onehot = (bids[None, :] == lax.broadcasted_iota(jnp.int32, (B, bm), 0)
              ).astype(jnp.float32)
    s_ref[...] += jnp.dot(onehot, h, preferred_element_type=jnp.float32)
    c_ref[...] += jnp.sum(onehot, axis=1)[:, None]

    @pl.when(i == pl.num_programs(0) - 1)
    def _():
        mean = s_ref[...] / jnp.maximum(c_ref[...], jnp.float32(1.0))
        o_ref[...] = (jnp.dot(mean, wfc_ref[...],
                              preferred_element_type=jnp.float32)
                      + bfc_ref[...])


def _final_layer(acc, den, bias, batch3, wfc, bfc, bm=1000):
    return pl.pallas_call(
        _final_body,
        grid=(N // bm,),
        in_specs=[pl.BlockSpec((NC, bm, H), lambda i: (0, i, 0)),
                  pl.BlockSpec((1, NW, bm), lambda i: (i, 0, 0)),
                  pl.BlockSpec((1, H), lambda i: (0, 0)),
                  pl.BlockSpec((1, 1, bm), lambda i: (i, 0, 0)),
                  pl.BlockSpec((H, O), lambda i: (0, 0)),
                  pl.BlockSpec((1, O), lambda i: (0, 0))],
        out_specs=pl.BlockSpec((B, O), lambda i: (0, 0)),
        out_shape=jax.ShapeDtypeStruct((B, O), jnp.float32),
        scratch_shapes=[pltpu.VMEM((B, H), jnp.float32),
                        pltpu.VMEM((B, 1), jnp.float32)],
    )(acc, den, bias, batch3, wfc, bfc)


def kernel(x, edge_index, edge_attr, batch, Wl1, Wr1, We1, att1, b1,
           Wl2, Wr2, We2, att2, b2, Wfc, bfc):
    src3 = edge_index[0].reshape(NW, NCHUNK, C)
    dst3 = edge_index[1].reshape(NW, NCHUNK, C)
    ei = jnp.stack([src3, dst3], axis=2)
    batch3 = batch.reshape(N // 1000, 1, 1000)

    xl1, xr1 = _mm2(x, jnp.concatenate([Wl1, Wr1], axis=1), bm=2000)
    e1, e2 = _mm2(edge_attr, jnp.concatenate([We1, We2], axis=1), bm=4000)

    acc1, den1 = _sc_edge_layer(xl1, xr1, e1, ei, att1)
    den1t = den1.reshape(NW, N // 1000, 1000).transpose(1, 0, 2)
    xl2, xr2 = _mid_layer(acc1, den1t, b1.reshape(1, H),
                          jnp.concatenate([Wl2, Wr2], axis=1))
    acc2, den2 = _sc_edge_layer(xl2, xr2, e2, ei, att2)
    den2t = den2.reshape(NW, N // 1000, 1000).transpose(1, 0, 2)
    return _final_layer(acc2, den2t, b2.reshape(1, H), batch3, Wfc,
                        bfc.reshape(1, O))


# speedup vs baseline: 2.0147x; 1.0000x over previous
import jax
import jax.numpy as jnp
from jax import lax
from jax.experimental import pallas as pl
from jax.experimental.pallas import tpu as pltpu
from jax.experimental.pallas import tpu_sc as plsc

N = 10000
E = 320000
D = 128
H = 128
O = 128
ED = 16
B = 64

NC = 2
NS = 16
NW = NC * NS
L = 16
K8 = H // L

EPW = E // NW
C = 40
NCHUNK = EPW // C
ZR = 40


def _sc_edge_body(xl_hbm, xr_hbm, e_hbm, ei_hbm, att_hbm,
                  acc_hbm, den_hbm,
                  acc_sp, xl_v, xr_v, en_v, idx_v, dsc_v, att_v,
                  hs4_v, den_v, sem_g, sem_i, sem_d, sem_s):
    cc = lax.axis_index("c")
    ss = lax.axis_index("s")
    wid = ss * NC + cc

    def _zrow(r, _):
        for k in range(K8):
            en_v[0][r, pl.ds(k * L, L)] = jnp.zeros((L,), jnp.float32)
        return _
    lax.fori_loop(0, ZR, _zrow, None)
    nblk = jnp.where(ss == NS - 1, 10, 16)
    row0 = ss * 640

    def _zblk(j, _):
        r0 = pl.multiple_of(row0 + j * ZR, ZR)
        pltpu.sync_copy(en_v[0], acc_sp.at[pl.ds(r0, ZR)])
        return _
    lax.fori_loop(0, nblk, _zblk, None)

    def _zden(r, _):
        den_v[pl.ds(r * L, L)] = jnp.zeros((L,), jnp.float32)
        return _
    lax.fori_loop(0, N // L, _zden, None)
    pltpu.sync_copy(att_hbm, att_v)
    plsc.subcore_barrier()

    att_regs = [att_v[pl.ds(k * L, L)] for k in range(K8)]
    lanes = lax.broadcasted_iota(jnp.int32, (L,), 0)
    lane0 = lanes == 0

    def _hsum(v, hs_v):
        for sh in (8, 4, 2, 1):
            hs_v[...] = v
            v = v + plsc.load_gather(hs_v, [lanes ^ sh])
        return v

    def _g_issue(b, ch):
        pltpu.async_copy(xl_hbm.at[idx_v[b].at[0]], xl_v[b], sem_g[b])
        pltpu.async_copy(xr_hbm.at[idx_v[b].at[1]], xr_v[b], sem_g[b])
        pltpu.async_copy(e_hbm.at[wid, ch], en_v[b], sem_g[b])

    def _compute(b):
        def _edge4(it, _):
            for j in range(4):
                i = it * 4 + j
                xl_regs = [xl_v[b][i, pl.ds(k * L, L)] for k in range(K8)]
                acc = None
                for k in range(K8):
                    z = (xl_regs[k] + xr_v[b][i, pl.ds(k * L, L)]
                         + en_v[b][i, pl.ds(k * L, L)])
                    lm = jnp.maximum(z, jnp.float32(0.2) * z)
                    t = lm * att_regs[k]
                    acc = t if acc is None else acc + t
                ex = jnp.exp(_hsum(acc, hs4_v[j]))
                for k in range(K8):
                    en_v[b][i, pl.ds(k * L, L)] = xl_regs[k] * ex
                dsp = plsc.load_gather(idx_v[b],
                                       [jnp.full((L,), 1, jnp.int32),
                                        lax.broadcast(i, (L,))])
                plsc.addupdate_scatter(den_v, [dsp], ex, mask=lane0)
            return _
        lax.fori_loop(0, C // 4, _edge4, None)

    pltpu.sync_copy(ei_hbm.at[wid, 0], idx_v[0])
    _g_issue(0, 0)
    pltpu.async_copy(ei_hbm.at[wid, 1], idx_v[1], sem_i[1])
    pltpu.async_copy(ei_hbm.at[wid, 0, 1], dsc_v[0], sem_d[0])

    def _iter(it, _):
        for b in (0, 1):
            ch = it * 2 + b
            nb = 1 - b
            pltpu.make_async_copy(e_hbm.at[0, 0], xl_v[b], sem_g[b]).wait()
            pltpu.make_async_copy(e_hbm.at[0, 0], xr_v[b], sem_g[b]).wait()
            pltpu.make_async_copy(e_hbm.at[0, 0], en_v[b], sem_g[b]).wait()

            @pl.when(ch + 1 < NCHUNK)
            def _():
                pltpu.make_async_copy(ei_hbm.at[0, 0], idx_v[nb],
                                      sem_i[nb]).wait()

                @pl.when(ch >= 1)
                def _():
                    pltpu.make_async_copy(e_hbm.at[0, 0], en_v[nb],
                                          sem_s[nb]).wait()

                _g_issue(nb, ch + 1)
                pltpu.async_copy(ei_hbm.at[wid, ch + 1, 1], dsc_v[nb],
                                 sem_d[nb])

            _compute(b)

            @pl.when(ch + 2 < NCHUNK)
            def _():
                pltpu.async_copy(ei_hbm.at[wid, ch + 2], idx_v[b], sem_i[b])

            pltpu.make_async_copy(ei_hbm.at[0, 0, 1], dsc_v[b],
                                  sem_d[b]).wait()
            pltpu.async_copy(en_v[b], acc_sp.at[dsc_v[b]], sem_s[b],
                             add=True)
        return _
    lax.fori_loop(0, NCHUNK // 2, _iter, None)

    pltpu.make_async_copy(e_hbm.at[0, 0], en_v[0], sem_s[0]).wait()
    pltpu.make_async_copy(e_hbm.at[0, 0], en_v[1], sem_s[1]).wait()

    plsc.subcore_barrier()

    def _xblk(j, _):
        r0 = pl.multiple_of(row0 + j * ZR, ZR)
        pltpu.sync_copy(acc_sp.at[pl.ds(r0, ZR)], acc_hbm.at[cc, pl.ds(r0, ZR)])
        return _
    lax.fori_loop(0, nblk, _xblk, None)
    pltpu.sync_copy(den_v, den_hbm.at[wid])


def _sc_edge_layer(xl, xr, e_emb, ei, att):
    e3 = e_emb.reshape(NW, NCHUNK, C, H)
    f = pl.kernel(
        _sc_edge_body,
        out_type=[jax.ShapeDtypeStruct((NC, N, H), jnp.float32),
                  jax.ShapeDtypeStruct((NW, N), jnp.float32)],
        mesh=plsc.VectorSubcoreMesh(core_axis_name="c", subcore_axis_name="s"),
        compiler_params=pltpu.CompilerParams(needs_layout_passes=False),
        scratch_types=[
            pltpu.VMEM_SHARED((N, H), jnp.float32),
            [pltpu.VMEM((C, H), jnp.float32) for _ in range(2)],
            [pltpu.VMEM((C, H), jnp.float32) for _ in range(2)],
            [pltpu.VMEM((C, H), jnp.float32) for _ in range(2)],
            [pltpu.VMEM((2, C), jnp.int32) for _ in range(2)],
            [pltpu.VMEM((C,), jnp.int32) for _ in range(2)],
            pltpu.VMEM((H,), jnp.float32),
            [pltpu.VMEM((L,), jnp.float32) for _ in range(4)],
            pltpu.VMEM((N,), jnp.float32),
            [pltpu.SemaphoreType.DMA for _ in range(2)],
            [pltpu.SemaphoreType.DMA for _ in range(2)],
            [pltpu.SemaphoreType.DMA for _ in range(2)],
            [pltpu.SemaphoreType.DMA for _ in range(2)],
        ],
    )
    return f(xl, xr, e3, ei, att)


def _mm2_body(x_ref, w_ref, o1_ref, o2_ref):
    r = jnp.dot(x_ref[...], w_ref[...], preferred_element_type=jnp.float32)
    o1_ref[...] = r[:, :H]
    o2_ref[...] = r[:, H:]


def _mm2(x, wcat, bm):
    m = x.shape[0]
    k = x.shape[1]
    return pl.pallas_call(
        _mm2_body,
        grid=(m // bm,),
        in_specs=[pl.BlockSpec((bm, k), lambda i: (i, 0)),
                  pl.BlockSpec((k, 2 * H), lambda i: (0, 0))],
        out_specs=[pl.BlockSpec((bm, H), lambda i: (i, 0)),
                   pl.BlockSpec((bm, H), lambda i: (i, 0))],
        out_shape=[jax.ShapeDtypeStruct((m, H), jnp.float32),
                   jax.ShapeDtypeStruct((m, H), jnp.float32)],
    )(x, wcat)


def _combine(acc_ref, den_ref, b_ref):
    a = acc_ref[0] + acc_ref[1]
    den = jnp.sum(den_ref[0], axis=0) + jnp.float32(1e-16)
    return jnp.maximum(a / den[:, None] + b_ref[...], jnp.float32(0.0))


def _mid_body(acc_ref, den_ref, b_ref, w_ref, o1_ref, o2_ref):
    h = _combine(acc_ref, den_ref, b_ref)
    r = jnp.dot(h, w_ref[...], preferred_element_type=jnp.float32)
    o1_ref[...] = r[:, :H]
    o2_ref[...] = r[:, H:]


def _mid_layer(acc, den, bias, wcat, bm=1000):
    return pl.pallas_call(
        _mid_body,
        grid=(N // bm,),
        in_specs=[pl.BlockSpec((NC, bm, H), lambda i: (0, i, 0)),
                  pl.BlockSpec((1, NW, bm), lambda i: (i, 0, 0)),
                  pl.BlockSpec((1, H), lambda i: (0, 0)),
                  pl.BlockSpec((H, 2 * H), lambda i: (0, 0))],
        out_specs=[pl.BlockSpec((bm, H), lambda i: (i, 0)),
                   pl.BlockSpec((bm, H), lambda i: (i, 0))],
        out_shape=[jax.ShapeDtypeStruct((N, H), jnp.float32),
                   jax.ShapeDtypeStruct((N, H), jnp.float32)],
    )(acc, den, bias, wcat)


def _final_body(acc_ref, den_ref, b_ref, batch_ref, wfc_ref, bfc_ref, o_ref,
                s_ref, c_ref):
    i = pl.program_id(0)

    @pl.when(i == 0)
    def _():
        s_ref[...] = jnp.zeros_like(s_ref)
        c_ref[...] = jnp.zeros_like(c_ref)

    h = _combine(acc_ref, den_ref, b_ref)
    bids = batch_ref[0, 0, :]
    bm = h.shape[0]
    onehot = (bids[None, :] == lax.broadcasted_iota(jnp.int32, (B, bm), 0)
              ).astype(jnp.float32)
    s_ref[...] += jnp.dot(onehot, h, preferred_element_type=jnp.float32)
    c_ref[...] += jnp.sum(onehot, axis=1)[:, None]

    @pl.when(i == pl.num_programs(0) - 1)
    def _():
        mean = s_ref[...] / jnp.maximum(c_ref[...], jnp.float32(1.0))
        o_ref[...] = (jnp.dot(mean, wfc_ref[...],
                              preferred_element_type=jnp.float32)
                      + bfc_ref[...])


def _final_layer(acc, den, bias, batch3, wfc, bfc, bm=1000):
    return pl.pallas_call(
        _final_body,
        grid=(N // bm,),
        in_specs=[pl.BlockSpec((NC, bm, H), lambda i: (0, i, 0)),
                  pl.BlockSpec((1, NW, bm), lambda i: (i, 0, 0)),
                  pl.BlockSpec((1, H), lambda i: (0, 0)),
                  pl.BlockSpec((1, 1, bm), lambda i: (i, 0, 0)),
                  pl.BlockSpec((H, O), lambda i: (0, 0)),
                  pl.BlockSpec((1, O), lambda i: (0, 0))],
        out_specs=pl.BlockSpec((B, O), lambda i: (0, 0)),
        out_shape=jax.ShapeDtypeStruct((B, O), jnp.float32),
        scratch_shapes=[pltpu.VMEM((B, H), jnp.float32),
                        pltpu.VMEM((B, 1), jnp.float32)],
    )(acc, den, bias, batch3, wfc, bfc)


def kernel(x, edge_index, edge_attr, batch, Wl1, Wr1, We1, att1, b1,
           Wl2, Wr2, We2, att2, b2, Wfc, bfc):
    src3 = edge_index[0].reshape(NW, NCHUNK, C)
    dst3 = edge_index[1].reshape(NW, NCHUNK, C)
    ei = jnp.stack([src3, dst3], axis=2)
    batch3 = batch.reshape(N // 1000, 1, 1000)

    xl1, xr1 = _mm2(x, jnp.concatenate([Wl1, Wr1], axis=1), bm=2000)
    e1, e2 = _mm2(edge_attr, jnp.concatenate([We1, We2], axis=1), bm=4000)

    acc1, den1 = _sc_edge_layer(xl1, xr1, e1, ei, att1)
    den1t = den1.reshape(NW, N // 1000, 1000).transpose(1, 0, 2)
    xl2, xr2 = _mid_layer(acc1, den1t, b1.reshape(1, H),
                          jnp.concatenate([Wl2, Wr2], axis=1))
    acc2, den2 = _sc_edge_layer(xl2, xr2, e2, ei, att2)
    den2t = den2.reshape(NW, N // 1000, 1000).transpose(1, 0, 2)
    return _final_layer(acc2, den2t, b2.reshape(1, H), batch3, Wfc,
                        bfc.reshape(1, O))


# speedup vs baseline: 2.0455x; 1.0153x over previous
import jax
import jax.numpy as jnp
from jax import lax
from jax.experimental import pallas as pl
from jax.experimental.pallas import tpu as pltpu
from jax.experimental.pallas import tpu_sc as plsc

N = 10000
E = 320000
D = 128
H = 128
O = 128
ED = 16
B = 64

NC = 2
NS = 16
NW = NC * NS
L = 16
K8 = H // L

EPW = E // NW
C = 40
NCHUNK = EPW // C
ZR = 40


def _sc_edge_body(xl_hbm, xr_hbm, e_hbm, ei_hbm, att_hbm,
                  acc_hbm, den_hbm,
                  acc_sp, xl_v, xr_v, en_v, idx_v, dsc_v, att_v,
                  hs4_v, den_v, sem_g, sem_i, sem_d, sem_s):
    cc = lax.axis_index("c")
    ss = lax.axis_index("s")
    wid = ss * NC + cc

    def _zrow(r, _):
        for k in range(K8):
            en_v[0][r, pl.ds(k * L, L)] = jnp.zeros((L,), jnp.float32)
        return _
    lax.fori_loop(0, ZR, _zrow, None)
    nblk = jnp.where(ss == NS - 1, 10, 16)
    row0 = ss * 640

    def _zblk(j, _):
        r0 = pl.multiple_of(row0 + j * ZR, ZR)
        pltpu.sync_copy(en_v[0], acc_sp.at[pl.ds(r0, ZR)])
        return _
    lax.fori_loop(0, nblk, _zblk, None)

    def _zden(r, _):
        den_v[pl.ds(r * L, L)] = jnp.zeros((L,), jnp.float32)
        return _
    lax.fori_loop(0, N // L, _zden, None)
    pltpu.sync_copy(att_hbm, att_v)
    plsc.subcore_barrier()

    att_regs = [att_v[pl.ds(k * L, L)] for k in range(K8)]
    lanes = lax.broadcasted_iota(jnp.int32, (L,), 0)
    lane0 = lanes == 0

    def _hsum(v, hs_v):
        for sh in (8, 4, 2, 1):
            hs_v[...] = v
            v = v + plsc.load_gather(hs_v, [lanes ^ sh])
        return v

    def _g_issue(b, ch):
        pltpu.async_copy(xl_hbm.at[idx_v[b].at[0]], xl_v[b], sem_g[b])
        pltpu.async_copy(xr_hbm.at[idx_v[b].at[1]], xr_v[b], sem_g[b])
        pltpu.async_copy(e_hbm.at[wid, ch], en_v[b], sem_g[b])

    def _compute(b):
        def _edge4(it, _):
            for j in range(4):
                i = it * 4 + j
                xl_regs = [xl_v[b][i, pl.ds(k * L, L)] for k in range(K8)]
                acc = None
                for k in range(K8):
                    z = (xl_regs[k] + xr_v[b][i, pl.ds(k * L, L)]
                         + en_v[b][i, pl.ds(k * L, L)])
                    lm = jnp.maximum(z, jnp.float32(0.2) * z)
                    t = lm * att_regs[k]
                    acc = t if acc is None else acc + t
                ex = jnp.exp(_hsum(acc, hs4_v[j]))
                for k in range(K8):
                    en_v[b][i, pl.ds(k * L, L)] = xl_regs[k] * ex
                dsp = plsc.load_gather(idx_v[b],
                                       [jnp.full((L,), 1, jnp.int32),
                                        lax.broadcast(i, (L,))])
                plsc.addupdate_scatter(den_v, [dsp], ex, mask=lane0)
            return _
        lax.fori_loop(0, C // 4, _edge4, None)

    pltpu.sync_copy(ei_hbm.at[wid, 0], idx_v[0])
    _g_issue(0, 0)
    pltpu.async_copy(ei_hbm.at[wid, 1], idx_v[1], sem_i[1])
    pltpu.async_copy(ei_hbm.at[wid, 0, 1], dsc_v[0], sem_d[0])

    def _iter(it, _):
        for b in (0, 1):
            ch = it * 2 + b
            nb = 1 - b
            pltpu.make_async_copy(e_hbm.at[0, 0], xl_v[b], sem_g[b]).wait()
            pltpu.make_async_copy(e_hbm.at[0, 0], xr_v[b], sem_g[b]).wait()
            pltpu.make_async_copy(e_hbm.at[0, 0], en_v[b], sem_g[b]).wait()

            @pl.when(ch + 1 < NCHUNK)
            def _():
                pltpu.make_async_copy(ei_hbm.at[0, 0], idx_v[nb],
                                      sem_i[nb]).wait()

                @pl.when(ch >= 1)
                def _():
                    pltpu.make_async_copy(e_hbm.at[0, 0], en_v[nb],
                                          sem_s[nb]).wait()

                _g_issue(nb, ch + 1)
                pltpu.async_copy(ei_hbm.at[wid, ch + 1, 1], dsc_v[nb],
                                 sem_d[nb])

            _compute(b)

            @pl.when(ch + 2 < NCHUNK)
            def _():
                pltpu.async_copy(ei_hbm.at[wid, ch + 2], idx_v[b], sem_i[b])

            pltpu.make_async_copy(ei_hbm.at[0, 0, 1], dsc_v[b],
                                  sem_d[b]).wait()
            pltpu.async_copy(en_v[b], acc_sp.at[dsc_v[b]], sem_s[b],
                             add=True)
        return _
    lax.fori_loop(0, NCHUNK // 2, _iter, None)

    pltpu.make_async_copy(e_hbm.at[0, 0], en_v[0], sem_s[0]).wait()
    pltpu.make_async_copy(e_hbm.at[0, 0], en_v[1], sem_s[1]).wait()

    plsc.subcore_barrier()

    def _xblk(j, _):
        r0 = pl.multiple_of(row0 + j * ZR, ZR)
        pltpu.sync_copy(acc_sp.at[pl.ds(r0, ZR)], acc_hbm.at[cc, pl.ds(r0, ZR)])
        return _
    lax.fori_loop(0, nblk, _xblk, None)
    pltpu.sync_copy(den_v, den_hbm.at[wid])


def _sc_edge_layer(xl, xr, e_emb, ei, att):
    e3 = e_emb.reshape(NW, NCHUNK, C, H)
    f = pl.kernel(
        _sc_edge_body,
        out_type=[jax.ShapeDtypeStruct((NC, N, H), jnp.float32),
                  jax.ShapeDtypeStruct((NW, N), jnp.float32)],
        mesh=plsc.VectorSubcoreMesh(core_axis_name="c", subcore_axis_name="s"),
        compiler_params=pltpu.CompilerParams(needs_layout_passes=False),
        scratch_types=[
            pltpu.VMEM_SHARED((N, H), jnp.float32),
            [pltpu.VMEM((C, H), jnp.float32) for _ in range(2)],
            [pltpu.VMEM((C, H), jnp.float32) for _ in range(2)],
            [pltpu.VMEM((C, H), jnp.float32) for _ in range(2)],
            [pltpu.VMEM((2, C), jnp.int32) for _ in range(2)],
            [pltpu.VMEM((C,), jnp.int32) for _ in range(2)],
            pltpu.VMEM((H,), jnp.float32),
            [pltpu.VMEM((L,), jnp.float32) for _ in range(4)],
            pltpu.VMEM((N,), jnp.float32),
            [pltpu.SemaphoreType.DMA for _ in range(2)],
            [pltpu.SemaphoreType.DMA for _ in range(2)],
            [pltpu.SemaphoreType.DMA for _ in range(2)],
            [pltpu.SemaphoreType.DMA for _ in range(2)],
        ],
    )
    return f(xl, xr, e3, ei, att)


def _mm1_body(x_ref, w_ref, o_ref):
    o_ref[...] = jnp.dot(x_ref[...], w_ref[...],
                         preferred_element_type=jnp.float32)


def _mm1(x, w, bm):
    m = x.shape[0]
    k = x.shape[1]
    return pl.pallas_call(
        _mm1_body,
        grid=(m // bm,),
        in_specs=[pl.BlockSpec((bm, k), lambda i: (i, 0)),
                  pl.BlockSpec((k, H), lambda i: (0, 0))],
        out_specs=pl.BlockSpec((bm, H), lambda i: (i, 0)),
        out_shape=jax.ShapeDtypeStruct((m, H), jnp.float32),
    )(x, w)


def _mm2_body(x_ref, w_ref, o1_ref, o2_ref):
    r = jnp.dot(x_ref[...], w_ref[...], preferred_element_type=jnp.float32)
    o1_ref[...] = r[:, :H]
    o2_ref[...] = r[:, H:]


def _mm2(x, wcat, bm):
    m = x.shape[0]
    k = x.shape[1]
    return pl.pallas_call(
        _mm2_body,
        grid=(m // bm,),
        in_specs=[pl.BlockSpec((bm, k), lambda i: (i, 0)),
                  pl.BlockSpec((k, 2 * H), lambda i: (0, 0))],
        out_specs=[pl.BlockSpec((bm, H), lambda i: (i, 0)),
                   pl.BlockSpec((bm, H), lambda i: (i, 0))],
        out_shape=[jax.ShapeDtypeStruct((m, H), jnp.float32),
                   jax.ShapeDtypeStruct((m, H), jnp.float32)],
    )(x, wcat)


def _combine(acc_ref, den_ref, b_ref):
    a = acc_ref[0] + acc_ref[1]
    den = jnp.sum(den_ref[0], axis=0) + jnp.float32(1e-16)
    return jnp.maximum(a / den[:, None] + b_ref[...], jnp.float32(0.0))


def _mid_body(acc_ref, den_ref, b_ref, w_ref, o1_ref, o2_ref):
    h = _combine(acc_ref, den_ref, b_ref)
    r = jnp.dot(h, w_ref[...], preferred_element_type=jnp.float32)
    o1_ref[...] = r[:, :H]
    o2_ref[...] = r[:, H:]


def _mid_layer(acc, den, bias, wcat, bm=1000):
    return pl.pallas_call(
        _mid_body,
        grid=(N // bm,),
        in_specs=[pl.BlockSpec((NC, bm, H), lambda i: (0, i, 0)),
                  pl.BlockSpec((1, NW, bm), lambda i: (i, 0, 0)),
                  pl.BlockSpec((1, H), lambda i: (0, 0)),
                  pl.BlockSpec((H, 2 * H), lambda i: (0, 0))],
        out_specs=[pl.BlockSpec((bm, H), lambda i: (i, 0)),
                   pl.BlockSpec((bm, H), lambda i: (i, 0))],
        out_shape=[jax.ShapeDtypeStruct((N, H), jnp.float32),
                   jax.ShapeDtypeStruct((N, H), jnp.float32)],
    )(acc, den, bias, wcat)


def _final_body(acc_ref, den_ref, b_ref, batch_ref, wfc_ref, bfc_ref, o_ref,
                s_ref, c_ref):
    i = pl.program_id(0)

    @pl.when(i == 0)
    def _():
        s_ref[...] = jnp.zeros_like(s_ref)
        c_ref[...] = jnp.zeros_like(c_ref)

    h = _combine(acc_ref, den_ref, b_ref)
    bids = batch_ref[0, 0, :]
    bm = h.shape[0]
    onehot = (bids[None, :] == lax.broadcasted_iota(jnp.int32, (B, bm), 0)
              ).astype(jnp.float32)
    s_ref[...] += jnp.dot(onehot, h, preferred_element_type=jnp.float32)
    c_ref[...] += jnp.sum(onehot, axis=1)[:, None]

    @pl.when(i == pl.num_programs(0) - 1)
    def _():
        mean = s_ref[...] / jnp.maximum(c_ref[...], jnp.float32(1.0))
        o_ref[...] = (jnp.dot(mean, wfc_ref[...],
                              preferred_element_type=jnp.float32)
                      + bfc_ref[...])


def _final_layer(acc, den, bias, batch3, wfc, bfc, bm=1000):
    return pl.pallas_call(
        _final_body,
        grid=(N // bm,),
        in_specs=[pl.BlockSpec((NC, bm, H), lambda i: (0, i, 0)),
                  pl.BlockSpec((1, NW, bm), lambda i: (i, 0, 0)),
                  pl.BlockSpec((1, H), lambda i: (0, 0)),
                  pl.BlockSpec((1, 1, bm), lambda i: (i, 0, 0)),
                  pl.BlockSpec((H, O), lambda i: (0, 0)),
                  pl.BlockSpec((1, O), lambda i: (0, 0))],
        out_specs=pl.BlockSpec((B, O), lambda i: (0, 0)),
        out_shape=jax.ShapeDtypeStruct((B, O), jnp.float32),
        scratch_shapes=[pltpu.VMEM((B, H), jnp.float32),
                        pltpu.VMEM((B, 1), jnp.float32)],
    )(acc, den, bias, batch3, wfc, bfc)


def kernel(x, edge_index, edge_attr, batch, Wl1, Wr1, We1, att1, b1,
           Wl2, Wr2, We2, att2, b2, Wfc, bfc):
    src3 = edge_index[0].reshape(NW, NCHUNK, C)
    dst3 = edge_index[1].reshape(NW, NCHUNK, C)
    ei = jnp.stack([src3, dst3], axis=2)
    batch3 = batch.reshape(N // 1000, 1, 1000)

    xl1, xr1 = _mm2(x, jnp.concatenate([Wl1, Wr1], axis=1), bm=2000)
    e1 = _mm1(edge_attr, We1, bm=4000)
    e2 = _mm1(edge_attr, We2, bm=4000)

    acc1, den1 = _sc_edge_layer(xl1, xr1, e1, ei, att1)
    den1t = den1.reshape(NW, N // 1000, 1000).transpose(1, 0, 2)
    xl2, xr2 = _mid_layer(acc1, den1t, b1.reshape(1, H),
                          jnp.concatenate([Wl2, Wr2], axis=1))
    acc2, den2 = _sc_edge_layer(xl2, xr2, e2, ei, att2)
    den2t = den2.reshape(NW, N // 1000, 1000).transpose(1, 0, 2)
    return _final_layer(acc2, den2t, b2.reshape(1, H), batch3, Wfc,
                        bfc.reshape(1, O))
